# baseline (speedup 1.0000x reference)
import functools

import jax
import jax.numpy as jnp
from jax.experimental import pallas as pl
from jax.experimental.pallas import tpu as pltpu
from jax.experimental.pallas import tpu_sc as plsc

B, K, D, TOPK = 1024, 100000, 1024, 100
BK = 2048
NBLK = (K + BK - 1) // BK
NW = 32
BPW = B // NW


def _sc_gather(keys, gt):
    mesh = plsc.VectorSubcoreMesh(core_axis_name="c", subcore_axis_name="s")

    @functools.partial(
        pl.kernel, mesh=mesh,
        out_type=jax.ShapeDtypeStruct((B, D), jnp.float32),
        scratch_types=[
            pltpu.VMEM((BPW,), jnp.int32),
            pltpu.VMEM((BPW, D), jnp.float32),
            pltpu.SemaphoreType.DMA,
        ],
    )
    def gather_kernel(table_hbm, idx_hbm, out_hbm, idx_v, rows_v, sem):
        wid = jax.lax.axis_index("s") * 2 + jax.lax.axis_index("c")
        base = wid * BPW
        pltpu.sync_copy(idx_hbm.at[pl.ds(base, BPW)], idx_v)
        pltpu.async_copy(table_hbm.at[idx_v], rows_v, sem).wait()
        pltpu.sync_copy(rows_v, out_hbm.at[pl.ds(base, BPW)])

    return gather_kernel(keys, gt)


def _prep_body(y_ref, nq_ref, g_ref, qb_ref, sgt_ref, tlo_ref):
    qb = (y_ref[...] / nq_ref[...]).astype(jnp.bfloat16)
    qb_ref[...] = qb
    g = g_ref[...]
    ng = jnp.sqrt(jnp.sum(g * g, axis=1, keepdims=True)) + 1e-12
    gn = (g / ng).astype(jnp.bfloat16)
    s = jax.lax.dot_general(qb, gn, (((1,), (1,)), ((), ())),
                            preferred_element_type=jnp.float32)
    mask = (jax.lax.broadcasted_iota(jnp.int32, (B, B), 0)
            == jax.lax.broadcasted_iota(jnp.int32, (B, B), 1))
    sgt = jnp.sum(jnp.where(mask, s, 0.0), axis=1, keepdims=True)
    sgt_ref[...] = sgt
    t = jax.lax.bitcast_convert_type(sgt, jnp.int32)
    tlo_i = jnp.where(sgt == 0.0, jnp.int32(-2147483647),
                      jnp.where(sgt > 0, t - 1, t + 1))
    tlo_ref[...] = jax.lax.bitcast_convert_type(tlo_i, jnp.float32)


def _main_body(qb_ref, kb_ref, gt_ref, sgt_ref, tlo_ref, out_ref,
               s0_ref, s1_ref, cnt_ref):
    i = pl.program_id(0)

    @pl.when(i == 0)
    def _():
        cnt_ref[...] = jnp.zeros_like(cnt_ref)

    def score(s_ref):
        kb = kb_ref[...]
        n = jnp.sqrt(jnp.sum(kb * kb, axis=1, keepdims=True)) + 1e-12
        kn = (kb / n).astype(jnp.bfloat16)
        s_ref[...] = jax.lax.dot_general(
            qb_ref[...], kn, (((1,), (1,)), ((), ())),
            preferred_element_type=jnp.float32)

    def count(s_ref):
        j = i - 1
        s = s_ref[...]
        col = jax.lax.broadcasted_iota(jnp.int32, (1, BK), 1)
        thr = jnp.where(col < gt_ref[...] - j * BK, tlo_ref[...],
                        sgt_ref[...])

        @pl.when(j < NBLK - 1)
        def _():
            hit = s > thr
            cnt_ref[...] += jnp.sum(hit.astype(jnp.float32), axis=1,
                                    keepdims=True)

        @pl.when(j == NBLK - 1)
        def _():
            valid = col < (K - j * BK)
            hit = (s > thr) & valid
            cnt_ref[...] += jnp.sum(hit.astype(jnp.float32), axis=1,
                                    keepdims=True)
            rank = cnt_ref[...] + 1.0
            rr = jnp.where(rank <= TOPK, 1.0 / rank, 0.0)
            out_ref[0, 0] = jnp.sum(rr) / B

    @pl.when(i % 2 == 0)
    def _():
        @pl.when(i < NBLK)
        def _():
            score(s0_ref)

        @pl.when(i > 0)
        def _():
            count(s1_ref)

    @pl.when(i % 2 == 1)
    def _():
        @pl.when(i < NBLK)
        def _():
            score(s1_ref)

        @pl.when(i > 0)
        def _():
            count(s0_ref)


def kernel(y_hat, keys, gt_idx):
    gt = gt_idx.astype(jnp.int32)
    gt2d = gt.reshape(B, 1)
    nq = jnp.linalg.norm(y_hat, axis=-1, keepdims=True) + 1e-12

    g = _sc_gather(keys, gt)

    qb, sgt, tlo = pl.pallas_call(
        _prep_body,
        out_shape=[
            jax.ShapeDtypeStruct((B, D), jnp.bfloat16),
            jax.ShapeDtypeStruct((B, 1), jnp.float32),
            jax.ShapeDtypeStruct((B, 1), jnp.float32),
        ],
    )(y_hat, nq, g)

    out = pl.pallas_call(
        _main_body,
        grid=(NBLK + 1,),
        in_specs=[
            pl.BlockSpec((B, D), lambda i: (0, 0)),
            pl.BlockSpec((BK, D), lambda i: (jnp.minimum(i, NBLK - 1), 0)),
            pl.BlockSpec((B, 1), lambda i: (0, 0)),
            pl.BlockSpec((B, 1), lambda i: (0, 0)),
            pl.BlockSpec((B, 1), lambda i: (0, 0)),
        ],
        out_specs=pl.BlockSpec(memory_space=pltpu.SMEM),
        out_shape=jax.ShapeDtypeStruct((1, 1), jnp.float32),
        scratch_shapes=[
            pltpu.VMEM((B, BK), jnp.float32),
            pltpu.VMEM((B, BK), jnp.float32),
            pltpu.VMEM((B, 1), jnp.float32),
        ],
    )(qb, keys, gt2d, sgt, tlo)

    return out[0, 0]


# speedup vs baseline: 1.1802x; 1.1802x over previous
import functools

import jax
import jax.numpy as jnp
from jax.experimental import pallas as pl
from jax.experimental.pallas import tpu as pltpu
from jax.experimental.pallas import tpu_sc as plsc

B, K, D, TOPK = 1024, 100000, 1024, 100
BK = 2048
NBLK = (K + BK - 1) // BK
NW = 32
BPW = B // NW


def _sc_gather(keys, gt):
    mesh = plsc.VectorSubcoreMesh(core_axis_name="c", subcore_axis_name="s")

    @functools.partial(
        pl.kernel, mesh=mesh,
        out_type=jax.ShapeDtypeStruct((B, D), jnp.float32),
        scratch_types=[
            pltpu.VMEM((BPW,), jnp.int32),
            pltpu.VMEM((BPW, D), jnp.float32),
            pltpu.SemaphoreType.DMA,
        ],
    )
    def gather_kernel(table_hbm, idx_hbm, out_hbm, idx_v, rows_v, sem):
        wid = jax.lax.axis_index("s") * 2 + jax.lax.axis_index("c")
        base = wid * BPW
        pltpu.sync_copy(idx_hbm.at[pl.ds(base, BPW)], idx_v)
        pltpu.async_copy(table_hbm.at[idx_v], rows_v, sem).wait()
        pltpu.sync_copy(rows_v, out_hbm.at[pl.ds(base, BPW)])

    return gather_kernel(keys, gt)


def _prep_body(y_ref, nq_ref, g_ref, qb_ref, sgt_ref, tlo_ref):
    qb = (y_ref[...] / nq_ref[...]).astype(jnp.bfloat16)
    qb_ref[...] = qb
    g = g_ref[...]
    ng = jnp.sqrt(jnp.sum(g * g, axis=1, keepdims=True)) + 1e-12
    gn = (g / ng).astype(jnp.bfloat16)
    s = jax.lax.dot_general(qb, gn, (((1,), (1,)), ((), ())),
                            preferred_element_type=jnp.float32)
    mask = (jax.lax.broadcasted_iota(jnp.int32, (B, B), 0)
            == jax.lax.broadcasted_iota(jnp.int32, (B, B), 1))
    sgt = jnp.sum(jnp.where(mask, s, 0.0), axis=1, keepdims=True)
    sgt_ref[...] = sgt
    t = jax.lax.bitcast_convert_type(sgt, jnp.int32)
    tlo_i = jnp.where(sgt == 0.0, jnp.int32(-2147483647),
                      jnp.where(sgt > 0, t - 1, t + 1))
    tlo_ref[...] = jax.lax.bitcast_convert_type(tlo_i, jnp.float32)


def _main_body(qb_ref, kb_ref, gt_ref, sgt_ref, tlo_ref, out_ref, cnt_ref):
    i = pl.program_id(0)

    @pl.when(i == 0)
    def _():
        cnt_ref[...] = jnp.zeros_like(cnt_ref)

    kb = kb_ref[...]
    n = jnp.sqrt(jnp.sum(kb * kb, axis=1, keepdims=True)) + 1e-12
    kn = (kb / n).astype(jnp.bfloat16)
    s = jax.lax.dot_general(qb_ref[...], kn, (((1,), (1,)), ((), ())),
                            preferred_element_type=jnp.float32)
    col = jax.lax.broadcasted_iota(jnp.int32, (1, BK), 1)
    thr = jnp.where(col < gt_ref[...] - i * BK, tlo_ref[...], sgt_ref[...])

    @pl.when(i < NBLK - 1)
    def _():
        hit = s > thr
        cnt_ref[...] += jnp.sum(hit.astype(jnp.float32), axis=1,
                                keepdims=True)

    @pl.when(i == NBLK - 1)
    def _():
        valid = col < (K - i * BK)
        hit = (s > thr) & valid
        cnt_ref[...] += jnp.sum(hit.astype(jnp.float32), axis=1,
                                keepdims=True)
        rank = cnt_ref[...] + 1.0
        rr = jnp.where(rank <= TOPK, 1.0 / rank, 0.0)
        out_ref[0, 0] = jnp.sum(rr) / B


def kernel(y_hat, keys, gt_idx):
    gt = gt_idx.astype(jnp.int32)
    gt2d = gt.reshape(B, 1)
    nq = jnp.linalg.norm(y_hat, axis=-1, keepdims=True) + 1e-12

    g = _sc_gather(keys, gt)

    qb, sgt, tlo = pl.pallas_call(
        _prep_body,
        out_shape=[
            jax.ShapeDtypeStruct((B, D), jnp.bfloat16),
            jax.ShapeDtypeStruct((B, 1), jnp.float32),
            jax.ShapeDtypeStruct((B, 1), jnp.float32),
        ],
    )(y_hat, nq, g)

    out = pl.pallas_call(
        _main_body,
        grid=(NBLK,),
        in_specs=[
            pl.BlockSpec((B, D), lambda i: (0, 0)),
            pl.BlockSpec((BK, D), lambda i: (i, 0)),
            pl.BlockSpec((B, 1), lambda i: (0, 0)),
            pl.BlockSpec((B, 1), lambda i: (0, 0)),
            pl.BlockSpec((B, 1), lambda i: (0, 0)),
        ],
        out_specs=pl.BlockSpec(memory_space=pltpu.SMEM),
        out_shape=jax.ShapeDtypeStruct((1, 1), jnp.float32),
        scratch_shapes=[pltpu.VMEM((B, 1), jnp.float32)],
    )(qb, keys, gt2d, sgt, tlo)

    return out[0, 0]
